# baseline (speedup 1.0000x reference)
import jax
from jax.experimental import pallas as pl
from jax.experimental.pallas import tpu as pltpu

_NCHUNK = 8


def _pipelined_copy(x_ref, o_ref, buf, in_sems, out_sems):
    rows = x_ref.shape[0]
    chunk = rows // _NCHUNK

    def in_copy(i):
        return pltpu.make_async_copy(
            x_ref.at[pl.ds(i * chunk, chunk)], buf.at[i], in_sems.at[i]
        )

    def out_copy(i):
        return pltpu.make_async_copy(
            buf.at[i], o_ref.at[pl.ds(i * chunk, chunk)], out_sems.at[i]
        )

    for i in range(_NCHUNK):
        in_copy(i).start()
    for i in range(_NCHUNK):
        in_copy(i).wait()
        out_copy(i).start()
    for i in range(_NCHUNK):
        out_copy(i).wait()


def kernel(x, adj, embed_table):
    del adj, embed_table
    rows, cols = x.shape
    return pl.pallas_call(
        _pipelined_copy,
        in_specs=[pl.BlockSpec(memory_space=pl.ANY)],
        out_specs=pl.BlockSpec(memory_space=pl.ANY),
        out_shape=jax.ShapeDtypeStruct(x.shape, x.dtype),
        scratch_shapes=[
            pltpu.VMEM((_NCHUNK, rows // _NCHUNK, cols), x.dtype),
            pltpu.SemaphoreType.DMA((_NCHUNK,)),
            pltpu.SemaphoreType.DMA((_NCHUNK,)),
        ],
    )(x)
